# Initial kernel scaffold; baseline (speedup 1.0000x reference)
import jax
import jax.numpy as jnp
from jax.experimental import pallas as pl


def kernel(positions, hidden_states, Wq, Wk, Wv, Wo, q_norm_scale, k_norm_scale, input_ln_scale, post_ln_scale, Wg, W_gate, W_up, W_down):
    raise NotImplementedError("write your pallas kernel here")


# speedup vs baseline: 1.2164x; 1.2164x over previous
import jax
import jax.numpy as jnp
from jax.experimental import pallas as pl

T = 2048
D = 1024
NH = 16
NKV = 4
HD = 128
E = 8
TOPK = 2
DFF = 768
EPS = 1e-6
THETA = 1000000.0
HALF = HD // 2

TB_M = 1024


def _rms_norm(x, scale, eps=EPS):
    var = jnp.mean(jnp.square(x), axis=-1, keepdims=True)
    return x * jax.lax.rsqrt(var + eps) * scale


def _apply_rope(x, positions):
    inv_freq = 1.0 / (THETA ** (jnp.arange(0, HALF, dtype=jnp.float32) / HALF))
    freqs = positions.astype(jnp.float32)[:, None] * inv_freq[None, :]
    cos = jnp.cos(freqs)[:, None, :]
    sin = jnp.sin(freqs)[:, None, :]
    x1 = x[..., :HALF]
    x2 = x[..., HALF:]
    return jnp.concatenate([x1 * cos - x2 * sin, x2 * cos + x1 * sin], axis=-1)


def _moe_kernel(h2_ref, cmb_ref, wg_ref, wu_ref, wd_ref, o_ref):
    e = pl.program_id(1)

    @pl.when(e == 0)
    def _init():
        o_ref[...] = jnp.zeros_like(o_ref)

    x = h2_ref[...]
    g = jax.lax.dot_general(x, wg_ref[0], (((1,), (0,)), ((), ())),
                            preferred_element_type=jnp.float32)
    u = jax.lax.dot_general(x, wu_ref[0], (((1,), (0,)), ((), ())),
                            preferred_element_type=jnp.float32)
    act = g * jax.nn.sigmoid(g) * u
    y = jax.lax.dot_general(act, wd_ref[0], (((1,), (0,)), ((), ())),
                            preferred_element_type=jnp.float32)
    lane = jax.lax.broadcasted_iota(jnp.int32, (TB_M, 128), 1)
    w = jnp.sum(jnp.where(lane == e, cmb_ref[...], 0.0), axis=-1,
                keepdims=True)
    o_ref[...] += w * y


def kernel(positions, hidden_states, Wq, Wk, Wv, Wo, q_norm_scale,
           k_norm_scale, input_ln_scale, post_ln_scale, Wg, W_gate, W_up,
           W_down):
    residual = hidden_states
    h = _rms_norm(hidden_states, input_ln_scale)
    q = (h @ Wq).reshape(T, NH, HD)
    k = (h @ Wk).reshape(T, NKV, HD)
    v = (h @ Wv).reshape(T, NKV, HD)
    q = _rms_norm(q, q_norm_scale)
    k = _rms_norm(k, k_norm_scale)
    q = _apply_rope(q, positions)
    k = _apply_rope(k, positions)
    rep = NH // NKV
    k = jnp.repeat(k, rep, axis=1)
    v = jnp.repeat(v, rep, axis=1)
    qh = q.transpose(1, 0, 2)
    kh = k.transpose(1, 0, 2)
    vh = v.transpose(1, 0, 2)
    scores = jnp.einsum('htd,hsd->hts', qh, kh) * (HD ** -0.5)
    causal = jnp.tril(jnp.ones((T, T), dtype=bool))
    scores = jnp.where(causal[None, :, :], scores, jnp.float32(-1e30))
    probs = jax.nn.softmax(scores, axis=-1)
    attn = jnp.einsum('hts,hsd->htd', probs, vh)
    attn = attn.transpose(1, 0, 2).reshape(T, NH * HD)
    attn_out = attn @ Wo
    h2 = attn_out + residual
    residual2 = h2
    hn = _rms_norm(h2, post_ln_scale)
    router_logits = hn @ Wg
    router_probs = jax.nn.softmax(router_logits.astype(jnp.float32), axis=-1)
    topk_w, topk_idx = jax.lax.top_k(router_probs, TOPK)
    topk_w = topk_w / jnp.sum(topk_w, axis=-1, keepdims=True)

    combine = jnp.zeros((T, 128), dtype=jnp.float32).at[
        jnp.arange(T)[:, None], topk_idx].add(topk_w)

    out = pl.pallas_call(
        _moe_kernel,
        grid=(T // TB_M, E),
        in_specs=[
            pl.BlockSpec((TB_M, D), lambda i, e: (i, 0)),
            pl.BlockSpec((TB_M, 128), lambda i, e: (i, 0)),
            pl.BlockSpec((1, D, DFF), lambda i, e: (e, 0, 0)),
            pl.BlockSpec((1, D, DFF), lambda i, e: (e, 0, 0)),
            pl.BlockSpec((1, DFF, D), lambda i, e: (e, 0, 0)),
        ],
        out_specs=pl.BlockSpec((TB_M, D), lambda i, e: (i, 0)),
        out_shape=jax.ShapeDtypeStruct((T, D), jnp.float32),
    )(hn, combine, W_gate, W_up, W_down)

    return (out, residual2)


# speedup vs baseline: 1.2328x; 1.0135x over previous
import functools

import jax
import jax.numpy as jnp
from jax import lax
from jax.experimental import pallas as pl
from jax.experimental.pallas import tpu as pltpu
from jax.experimental.pallas import tpu_sc as plsc

T = 2048
D = 1024
NH = 16
NKV = 4
HD = 128
E = 8
TOPK = 2
DFF = 768
EPS = 1e-6
THETA = 1000000.0
HALF = HD // 2

R = 128
NP = 4096 + 1024
NT = NP // R
PADMAX = NP - T * TOPK
NW = 32
TW = T // NW
PW = PADMAX // NW
TB_CMB = 512


def _rms_norm(x, scale, eps=EPS):
    var = jnp.mean(jnp.square(x), axis=-1, keepdims=True)
    return x * jax.lax.rsqrt(var + eps) * scale


def _apply_rope(x, positions):
    inv_freq = 1.0 / (THETA ** (jnp.arange(0, HALF, dtype=jnp.float32) / HALF))
    freqs = positions.astype(jnp.float32)[:, None] * inv_freq[None, :]
    cos = jnp.cos(freqs)[:, None, :]
    sin = jnp.sin(freqs)[:, None, :]
    x1 = x[..., :HALF]
    x2 = x[..., HALF:]
    return jnp.concatenate([x1 * cos - x2 * sin, x2 * cos + x1 * sin], axis=-1)


def _dispatch_kernel(eidx_ref, ppos_ref, padpos_ref, te_ref):
    eidx = eidx_ref[...]
    lane = lax.broadcasted_iota(jnp.int32, (T, 128), 1)
    e0 = eidx[:, 0:1]
    e1 = eidx[:, 1:2]
    oh = ((lane == e0) | (lane == e1)).astype(jnp.float32)

    rowi = lax.broadcasted_iota(jnp.int32, (T, T), 0)
    colj = lax.broadcasted_iota(jnp.int32, (T, T), 1)
    tril = (rowi >= colj).astype(jnp.float32)
    cum = jax.lax.dot_general(tril, oh, (((1,), (0,)), ((), ())),
                              preferred_element_type=jnp.float32)
    counts = cum[T - 1:T, :]
    cnt_i = counts.astype(jnp.int32)
    pc_i = ((cnt_i + (R - 1)) // R) * R
    pc = pc_i.astype(jnp.float32)

    li = lax.broadcasted_iota(jnp.int32, (128, 128), 0)
    lj = lax.broadcasted_iota(jnp.int32, (128, 128), 1)
    ltri = (li <= lj).astype(jnp.float32)
    incl_pad = jax.lax.dot_general(pc, ltri, (((1,), (0,)), ((), ())),
                                   preferred_element_type=jnp.float32)
    off = incl_pad - pc

    cum_excl = cum - oh
    off_sel0 = jnp.sum(jnp.where(lane == e0, off, 0.0), axis=1, keepdims=True)
    rank0 = jnp.sum(jnp.where(lane == e0, cum_excl, 0.0), axis=1,
                    keepdims=True)
    pos0 = (off_sel0 + rank0).astype(jnp.int32)
    off_sel1 = jnp.sum(jnp.where(lane == e1, off, 0.0), axis=1, keepdims=True)
    rank1 = jnp.sum(jnp.where(lane == e1, cum_excl, 0.0), axis=1,
                    keepdims=True)
    pos1 = (off_sel1 + rank1).astype(jnp.int32)
    ppos_ref[...] = (jnp.where(lane == 0, pos0, 0)
                     + jnp.where(lane == 1, pos1, 0))

    padc = pc - counts
    incl_padc = jax.lax.dot_general(padc, ltri, (((1,), (0,)), ((), ())),
                                    preferred_element_type=jnp.float32)
    excl_padc = incl_padc - padc
    total_pad = incl_padc[0:1, E - 1:E]
    krow = lax.broadcasted_iota(jnp.int32, (PADMAX // 128, 128), 0)
    kcol = lax.broadcasted_iota(jnp.int32, (PADMAX // 128, 128), 1)
    kf = (krow * 128 + kcol).astype(jnp.float32)
    ek = jnp.zeros_like(krow)
    for e in range(E):
        ek = ek + (kf >= incl_padc[0:1, e:e + 1]).astype(jnp.int32)
    ek = jnp.minimum(ek, E - 1)
    offsel = jnp.zeros_like(kf)
    csel = jnp.zeros_like(kf)
    exsel = jnp.zeros_like(kf)
    for e in range(E):
        m = ek == e
        offsel = offsel + jnp.where(m, off[0:1, e:e + 1], 0.0)
        csel = csel + jnp.where(m, counts[0:1, e:e + 1], 0.0)
        exsel = exsel + jnp.where(m, excl_padc[0:1, e:e + 1], 0.0)
    pos_in = offsel + csel + (kf - exsel)
    pos_ovf = float(T * TOPK) + kf
    padpos_ref[...] = jnp.where(kf < total_pad, pos_in, pos_ovf).astype(
        jnp.int32)

    tif = (lax.broadcasted_iota(jnp.int32, (1, 128), 1) * R).astype(
        jnp.float32)
    te = jnp.zeros((1, 128), jnp.int32)
    for e in range(E):
        te = te + (tif >= incl_pad[0:1, e:e + 1]).astype(jnp.int32)
    te_ref[...] = jnp.minimum(te, E - 1)


def _sc_mesh():
    return plsc.VectorSubcoreMesh(core_axis_name="c", subcore_axis_name="s")


def _wid():
    info = plsc.get_sparse_core_info()
    return lax.axis_index("s") * info.num_cores + lax.axis_index("c")


def _sc_scatter_body(h2, pos0, pos1, padpos, zrows_hbm, x_out,
                     idx_v, rows_v, zrows_v, pidx_v, sem):
    w = _wid()
    tbase = w * TW
    pltpu.sync_copy(h2.at[pl.ds(tbase, TW)], rows_v)
    pltpu.sync_copy(pos0.at[pl.ds(tbase, TW)], idx_v)
    pltpu.async_copy(rows_v, x_out.at[idx_v], sem).wait()
    pltpu.sync_copy(pos1.at[pl.ds(tbase, TW)], idx_v)
    pltpu.async_copy(rows_v, x_out.at[idx_v], sem).wait()
    pltpu.sync_copy(padpos.at[pl.ds(w * PW, PW)], pidx_v)
    pltpu.sync_copy(zrows_hbm, zrows_v)
    pltpu.async_copy(zrows_v, x_out.at[pidx_v], sem).wait()


def _build_sorted(hn, pos0, pos1, padflat):
    zrows = jnp.zeros((PW, D), jnp.float32)
    f = pl.kernel(
        _sc_scatter_body,
        mesh=_sc_mesh(),
        out_type=jax.ShapeDtypeStruct((NP, D), jnp.float32),
        scratch_types=[
            pltpu.VMEM((TW,), jnp.int32),
            pltpu.VMEM((TW, D), jnp.float32),
            pltpu.VMEM((PW, D), jnp.float32),
            pltpu.VMEM((PW,), jnp.int32),
            pltpu.SemaphoreType.DMA,
        ],
    )
    return f(hn, pos0, pos1, padflat, zrows)


def _sc_gather_body(y, pos0, pos1, y0_out, y1_out, idx_v, rows_v, sem):
    w = _wid()
    tbase = w * TW
    pltpu.sync_copy(pos0.at[pl.ds(tbase, TW)], idx_v)
    pltpu.async_copy(y.at[idx_v], rows_v, sem).wait()
    pltpu.sync_copy(rows_v, y0_out.at[pl.ds(tbase, TW)])
    pltpu.sync_copy(pos1.at[pl.ds(tbase, TW)], idx_v)
    pltpu.async_copy(y.at[idx_v], rows_v, sem).wait()
    pltpu.sync_copy(rows_v, y1_out.at[pl.ds(tbase, TW)])


def _gather_pair(Y, pos0, pos1):
    f = pl.kernel(
        _sc_gather_body,
        mesh=_sc_mesh(),
        out_type=[
            jax.ShapeDtypeStruct((T, D), jnp.float32),
            jax.ShapeDtypeStruct((T, D), jnp.float32),
        ],
        scratch_types=[
            pltpu.VMEM((TW,), jnp.int32),
            pltpu.VMEM((TW, D), jnp.float32),
            pltpu.SemaphoreType.DMA,
        ],
    )
    return f(Y, pos0, pos1)


def _gffn_kernel(te_ref, x_ref, wg_ref, wu_ref, wd_ref, y_ref):
    x = x_ref[...]
    g = jax.lax.dot_general(x, wg_ref[0], (((1,), (0,)), ((), ())),
                            preferred_element_type=jnp.float32)
    u = jax.lax.dot_general(x, wu_ref[0], (((1,), (0,)), ((), ())),
                            preferred_element_type=jnp.float32)
    act = g * jax.nn.sigmoid(g) * u
    y_ref[...] = jax.lax.dot_general(act, wd_ref[0], (((1,), (0,)), ((), ())),
                                     preferred_element_type=jnp.float32)


def _comb_kernel(y0_ref, y1_ref, w0_ref, w1_ref, o_ref):
    o_ref[...] = w0_ref[...] * y0_ref[...] + w1_ref[...] * y1_ref[...]


def kernel(positions, hidden_states, Wq, Wk, Wv, Wo, q_norm_scale,
           k_norm_scale, input_ln_scale, post_ln_scale, Wg, W_gate, W_up,
           W_down):
    residual = hidden_states
    h = _rms_norm(hidden_states, input_ln_scale)
    q = (h @ Wq).reshape(T, NH, HD)
    k = (h @ Wk).reshape(T, NKV, HD)
    v = (h @ Wv).reshape(T, NKV, HD)
    q = _rms_norm(q, q_norm_scale)
    k = _rms_norm(k, k_norm_scale)
    q = _apply_rope(q, positions)
    k = _apply_rope(k, positions)
    rep = NH // NKV
    k = jnp.repeat(k, rep, axis=1)
    v = jnp.repeat(v, rep, axis=1)
    qh = q.transpose(1, 0, 2)
    kh = k.transpose(1, 0, 2)
    vh = v.transpose(1, 0, 2)
    scores = jnp.einsum('htd,hsd->hts', qh, kh) * (HD ** -0.5)
    causal = jnp.tril(jnp.ones((T, T), dtype=bool))
    scores = jnp.where(causal[None, :, :], scores, jnp.float32(-1e30))
    probs = jax.nn.softmax(scores, axis=-1)
    attn = jnp.einsum('hts,hsd->htd', probs, vh)
    attn = attn.transpose(1, 0, 2).reshape(T, NH * HD)
    attn_out = attn @ Wo
    h2 = attn_out + residual
    residual2 = h2
    hn = _rms_norm(h2, post_ln_scale)
    router_logits = hn @ Wg
    router_probs = jax.nn.softmax(router_logits.astype(jnp.float32), axis=-1)
    topk_w, topk_idx = jax.lax.top_k(router_probs, TOPK)
    topk_w = topk_w / jnp.sum(topk_w, axis=-1, keepdims=True)

    eidx_pad = jnp.pad(topk_idx, ((0, 0), (0, 128 - TOPK)))
    ppos, padpos, te128 = pl.pallas_call(
        _dispatch_kernel,
        grid=(1,),
        in_specs=[pl.BlockSpec((T, 128), lambda i: (0, 0))],
        out_specs=[
            pl.BlockSpec((T, 128), lambda i: (0, 0)),
            pl.BlockSpec((PADMAX // 128, 128), lambda i: (0, 0)),
            pl.BlockSpec((1, 128), lambda i: (0, 0)),
        ],
        out_shape=[
            jax.ShapeDtypeStruct((T, 128), jnp.int32),
            jax.ShapeDtypeStruct((PADMAX // 128, 128), jnp.int32),
            jax.ShapeDtypeStruct((1, 128), jnp.int32),
        ],
    )(eidx_pad)
    pos0 = ppos[:, 0]
    pos1 = ppos[:, 1]
    padflat = padpos.reshape(PADMAX)
    te = te128[0, :NT]

    X = _build_sorted(hn, pos0, pos1, padflat)

    Y = pl.pallas_call(
        _gffn_kernel,
        grid_spec=pltpu.PrefetchScalarGridSpec(
            num_scalar_prefetch=1,
            grid=(NT,),
            in_specs=[
                pl.BlockSpec((R, D), lambda i, te_r: (i, 0)),
                pl.BlockSpec((1, D, DFF), lambda i, te_r: (te_r[i], 0, 0)),
                pl.BlockSpec((1, D, DFF), lambda i, te_r: (te_r[i], 0, 0)),
                pl.BlockSpec((1, DFF, D), lambda i, te_r: (te_r[i], 0, 0)),
            ],
            out_specs=pl.BlockSpec((R, D), lambda i, te_r: (i, 0)),
        ),
        out_shape=jax.ShapeDtypeStruct((NP, D), jnp.float32),
    )(te, X, W_gate, W_up, W_down)

    Y0, Y1 = _gather_pair(Y, pos0, pos1)

    out = pl.pallas_call(
        _comb_kernel,
        grid=(T // TB_CMB,),
        in_specs=[
            pl.BlockSpec((TB_CMB, D), lambda i: (i, 0)),
            pl.BlockSpec((TB_CMB, D), lambda i: (i, 0)),
            pl.BlockSpec((TB_CMB, 1), lambda i: (i, 0)),
            pl.BlockSpec((TB_CMB, 1), lambda i: (i, 0)),
        ],
        out_specs=pl.BlockSpec((TB_CMB, D), lambda i: (i, 0)),
        out_shape=jax.ShapeDtypeStruct((T, D), jnp.float32),
    )(Y0, Y1, topk_w[:, 0:1], topk_w[:, 1:2])

    return (out, residual2)
